# Initial kernel scaffold; baseline (speedup 1.0000x reference)
import jax
import jax.numpy as jnp
from jax.experimental import pallas as pl


def kernel(matrix):
    raise NotImplementedError("write your pallas kernel here")


# speedup vs baseline: 1.4273x; 1.4273x over previous
import jax
import jax.numpy as jnp
from jax import lax
from jax.experimental import pallas as pl
from jax.experimental.pallas import tpu as pltpu

B = 1024
F = 4096
RB = 128
CB = 512
K_DROP = 103
N_SEL = 8


def _rowsq_body(m_ref, out_ref):
    x = m_ref[...]
    out_ref[...] = jnp.sum(x * x, axis=1, keepdims=True)


def _score_body(a_ref, b_ref, sqc_ref, sqr_ref, out_ref, dT_ref):
    j = pl.program_id(1)
    a = a_ref[...]
    b = b_ref[...]
    g = lax.dot_general(b, a, (((1,), (1,)), ((), ())),
                        preferred_element_type=jnp.float32)
    d2 = sqc_ref[...] + sqr_ref[0] - 2.0 * g
    d2 = jnp.maximum(d2, 0.0)
    safe = jnp.where(d2 > 0.0, d2, 1.0)
    d = jnp.where(d2 > 0.0, jnp.sqrt(safe), 0.0)
    dT_ref[pl.ds(j * RB, RB), :] = d

    @pl.when(j == (B // RB) - 1)
    def _select():
        dall = dT_ref[...]
        bits = lax.bitcast_convert_type(dall, jnp.int32)
        rowsum = jnp.sum(dall, axis=0, keepdims=True)
        rowmin = jnp.min(dall, axis=0, keepdims=True)

        def bit_step(t, T):
            bit = jnp.int32(30) - t
            cand = T | (jnp.int32(1) << bit)
            ge = bits >= cand
            cnt = jnp.sum(jnp.where(ge, 1, 0), axis=0, keepdims=True)
            return jnp.where(cnt >= K_DROP, cand, T)

        T = lax.fori_loop(0, 31, bit_step, jnp.zeros((1, RB), jnp.int32))
        gt = bits > T
        cnt_gt = jnp.sum(jnp.where(gt, 1.0, 0.0), axis=0, keepdims=True)
        sum_gt = jnp.sum(jnp.where(gt, dall, 0.0), axis=0, keepdims=True)
        kth = lax.bitcast_convert_type(T, jnp.float32)
        sumtop = sum_gt + (K_DROP - cnt_gt) * kth
        out_ref[0] = rowsum - sumtop - rowmin


def _select_body(scores_ref, m_ref, out_ref, w_ref):
    c = pl.program_id(0)

    @pl.when(c == 0)
    def _weights():
        s = scores_ref[...]
        iota = lax.broadcasted_iota(jnp.int32, (B, 1), 0)
        w = jnp.zeros((B, 1), jnp.float32)

        def pick(_, carry):
            s, w = carry
            m = jnp.min(s)
            elig = s == m
            idx = jnp.min(jnp.where(elig, iota, jnp.int32(2 ** 30)))
            onehot = iota == idx
            w = w + jnp.where(onehot, 1.0 / N_SEL, 0.0)
            s = jnp.where(onehot, jnp.float32(jnp.inf), s)
            return s, w

        _, w = lax.fori_loop(0, N_SEL, pick, (s, w))
        w_ref[...] = w

    out_ref[0] = jnp.sum(m_ref[...] * w_ref[...], axis=0, keepdims=True)


def kernel(matrix):
    rowsq = pl.pallas_call(
        _rowsq_body,
        grid=(B // RB,),
        in_specs=[pl.BlockSpec((RB, F), lambda i: (i, 0))],
        out_specs=pl.BlockSpec((RB, 1), lambda i: (i, 0)),
        out_shape=jax.ShapeDtypeStruct((B, 1), jnp.float32),
    )(matrix)

    sq_row3 = rowsq.reshape(B // RB, 1, RB)

    scores3 = pl.pallas_call(
        _score_body,
        grid=(B // RB, B // RB),
        in_specs=[
            pl.BlockSpec((RB, F), lambda i, j: (i, 0)),
            pl.BlockSpec((RB, F), lambda i, j: (j, 0)),
            pl.BlockSpec((RB, 1), lambda i, j: (j, 0)),
            pl.BlockSpec((1, 1, RB), lambda i, j: (i, 0, 0)),
        ],
        out_specs=pl.BlockSpec((1, 1, RB), lambda i, j: (i, 0, 0)),
        out_shape=jax.ShapeDtypeStruct((B // RB, 1, RB), jnp.float32),
        scratch_shapes=[pltpu.VMEM((B, RB), jnp.float32)],
    )(matrix, matrix, rowsq, sq_row3)

    scores = scores3.reshape(B, 1)

    out3 = pl.pallas_call(
        _select_body,
        grid=(F // CB,),
        in_specs=[
            pl.BlockSpec((B, 1), lambda c: (0, 0)),
            pl.BlockSpec((B, CB), lambda c: (0, c)),
        ],
        out_specs=pl.BlockSpec((1, 1, CB), lambda c: (0, 0, c)),
        out_shape=jax.ShapeDtypeStruct((1, 1, F // CB * CB), jnp.float32),
        scratch_shapes=[pltpu.VMEM((B, 1), jnp.float32)],
    )(scores, matrix)

    return out3.reshape(F)
